# baseline (speedup 1.0000x reference)
import functools

import jax
import jax.numpy as jnp
from jax import lax
from jax.experimental import pallas as pl
from jax.experimental.pallas import tpu as pltpu
from jax.experimental.pallas import tpu_sc as plsc

B = 16384
R_IN = 26
R_OUT = 11
D = 64
RUNS = ((0, 2, 0), (6, 1, 2), (12, 4, 3), (17, 1, 7), (20, 3, 8))

NC, NS = 2, 16
NW = NC * NS
BPW = B // NW
NB = 64
NCHUNK = BPW // NB

W_IN = R_IN * D
W_OUT = R_OUT * D


def _sc_body(in_hbm, out_hbm, buf, rsem, wsem0, wsem1):
    wid = lax.axis_index("s") * NC + lax.axis_index("c")
    base = wid * BPW
    wsems = (wsem0, wsem1)
    pending_write = [None, None]
    for c in range(NCHUNK):
        slot = c % 2
        b0 = base + c * NB
        if pending_write[slot] is not None:
            pending_write[slot].wait()
        reads = [
            pltpu.async_copy(
                in_hbm.at[pl.ds(b0, NB), pl.ds(src * D, w * D)],
                buf.at[slot, :, pl.ds(dst * D, w * D)],
                rsem,
            )
            for (src, w, dst) in RUNS
        ]
        for h in reads:
            h.wait()
        pending_write[slot] = pltpu.async_copy(
            buf.at[slot], out_hbm.at[pl.ds(b0, NB)], wsems[slot]
        )
    for h in pending_write:
        if h is not None:
            h.wait()


@jax.jit
def kernel(inputs):
    in2 = inputs.reshape(B, W_IN)
    mesh = plsc.VectorSubcoreMesh(core_axis_name="c", subcore_axis_name="s")
    out2 = pl.kernel(
        _sc_body,
        out_type=jax.ShapeDtypeStruct((B, W_OUT), jnp.float32),
        mesh=mesh,
        scratch_types=[
            pltpu.VMEM((2, NB, W_OUT), jnp.float32),
            pltpu.SemaphoreType.DMA,
            pltpu.SemaphoreType.DMA,
            pltpu.SemaphoreType.DMA,
        ],
        compiler_params=pltpu.CompilerParams(use_tc_tiling_on_sc=False),
    )(in2)
    return out2.reshape(B, R_OUT, D)
